# baseline (speedup 1.0000x reference)
import dataclasses
import functools

import jax
import jax.numpy as jnp
from jax import lax
from jax.experimental import pallas as pl
from jax.experimental.pallas import tpu as pltpu
from jax.experimental.pallas import tpu_sc as plsc

N = 10000
E = 320000
D = 128
H = 64
NCORE = 2
NSUB = 16
IW = 128
SROWS = E // IW
DROWS = SROWS
EP = 327680
SROWSP = EP // IW
MROWSP = SROWSP // NCORE
TROWS = MROWSP // NSUB
NBLK = TROWS // 8
NP = 10240
RPT = NP // NSUB

_mesh = plsc.VectorSubcoreMesh(core_axis_name="c", subcore_axis_name="s")
_f32 = jnp.float32


DTROWS = SROWSP // NSUB
HR = NP // IW
RED = HR // 8

_deg_cp = pltpu.CompilerParams()
if "needs_layout_passes" in pltpu.CompilerParams.__dataclass_fields__:
    _deg_cp = dataclasses.replace(_deg_cp, needs_layout_passes=False)

_DEG_KW = dict(
    mesh=_mesh,
    compiler_params=_deg_cp,
    out_type=jax.ShapeDtypeStruct((NCORE, HR, IW), _f32),
    scratch_types=[
        pltpu.VMEM((DTROWS, IW), jnp.int32),
        pltpu.VMEM((HR, IW), _f32),
        pltpu.VMEM((8, IW), _f32),
        pltpu.VMEM((8, IW), _f32),
        pltpu.VMEM((16,), _f32),
        pltpu.VMEM_SHARED((NSUB * HR, IW), _f32),
    ],
)


def _deg_body(ei2_hbm, z128_hbm, o128_hbm, out_hbm,
              idx_v, hist_v, acc_v, tmp_v, one_v, hist_sh):
    c = lax.axis_index("c")
    t = lax.axis_index("s")

    pltpu.sync_copy(z128_hbm.at[pl.ds(0, HR)], hist_v)
    pltpu.sync_copy(ei2_hbm.at[pl.ds(c * SROWSP + t * DTROWS, DTROWS)], idx_v)
    pltpu.sync_copy(o128_hbm.at[0, pl.ds(0, 16)], one_v)
    ones16 = one_v[...]

    @pl.loop(0, DTROWS)
    def _(j):
        for l in range(IW // 16):
            idx16 = idx_v[j, pl.ds(l * 16, 16)]
            ri = lax.shift_right_logical(idx16, 7)
            ci = lax.bitwise_and(idx16, 127)
            plsc.addupdate_scatter(hist_v, [ri, ci], ones16)

    pltpu.sync_copy(hist_v, hist_sh.at[pl.ds(t * HR, HR)])
    plsc.subcore_barrier()

    @pl.when(t < RED)
    def _():
        pltpu.sync_copy(hist_sh.at[pl.ds(t * 8, 8)], acc_v)

        @pl.loop(1, NSUB)
        def _(h):
            pltpu.sync_copy(hist_sh.at[pl.ds(h * HR + t * 8, 8)], tmp_v)
            for i in range(8):
                for l in range(IW // 16):
                    sl = (i, pl.ds(l * 16, 16))
                    acc_v[sl] = acc_v[sl] + tmp_v[sl]

        pltpu.sync_copy(acc_v, out_hbm.at[c, pl.ds(t * 8, 8)])


_deg_kernel = pl.kernel(_deg_body, **_DEG_KW)


_MP_KW = dict(
    mesh=_mesh,
    out_type=jax.ShapeDtypeStruct((NCORE, NP, D), _f32),
    scratch_types=[
        pltpu.VMEM((8, IW), jnp.int32),
        pltpu.VMEM((8, IW), jnp.int32),
        pltpu.VMEM((8, IW), jnp.int32),
        pltpu.VMEM((8, IW), jnp.int32),
        pltpu.VMEM((IW,), jnp.int32),
        pltpu.VMEM((IW, D), _f32),
        pltpu.VMEM((IW, D), _f32),
        pltpu.VMEM_SHARED((NP, D), _f32),
        pltpu.SemaphoreType.DMA,
        pltpu.SemaphoreType.DMA,
        pltpu.SemaphoreType.DMA,
        pltpu.SemaphoreType.DMA,
        pltpu.SemaphoreType.DMA,
        pltpu.SemaphoreType.DMA,
    ],
)


def _mp_body(tab_hbm, ei2_hbm, z128_hbm, out_hbm,
             sA, dA, sB, dB, pI, rA, rB, agg_sh,
             siA, siB, sgA, sgB, ssA, ssB):
    c = lax.axis_index("c")
    t = lax.axis_index("s")

    pltpu.sync_copy(z128_hbm, agg_sh.at[pl.ds(t * RPT, RPT)])
    pltpu.sync_copy(ei2_hbm.at[pl.ds(SROWSP - 1, 1)].at[0], pI)
    plsc.subcore_barrier()

    base = c * MROWSP + t * TROWS

    def idx_start(k, s_ref, d_ref, sem):
        r0 = base + 8 * k
        pltpu.async_copy(ei2_hbm.at[pl.ds(r0, 8)], s_ref, sem)
        pltpu.async_copy(ei2_hbm.at[pl.ds(SROWSP + r0, 8)], d_ref, sem)

    def idx_wait(s_ref, d_ref, sem):
        pltpu.make_async_copy(ei2_hbm.at[pl.ds(0, 8)], s_ref, sem).wait()
        pltpu.make_async_copy(ei2_hbm.at[pl.ds(0, 8)], d_ref, sem).wait()

    def g_start(s_ref, j, r_ref, sem):
        pltpu.async_copy(tab_hbm.at[s_ref.at[j]], r_ref, sem)

    def g_wait(s_ref, j, r_ref, sem):
        pltpu.make_async_copy(tab_hbm.at[s_ref.at[j]], r_ref, sem).wait()

    def s_start(r_ref, d_ref, j, sem):
        pltpu.async_copy(r_ref, agg_sh.at[d_ref.at[j]], sem, add=True)

    def s_wait(r_ref, d_ref, j, sem):
        pltpu.make_async_copy(r_ref, agg_sh.at[d_ref.at[j]], sem).wait()

    def do_block(k, sC, dC, siC, sN, dN, siN):
        for j in range(8):
            if j % 2 == 0:
                rX, sgX, ssX = rA, sgA, ssA
                rY, sgY, ssY = rB, sgB, ssB
            else:
                rX, sgX, ssX = rB, sgB, ssB
                rY, sgY, ssY = rA, sgA, ssA
            g_wait(sC, j, rX, sgX)
            s_start(rX, dC, j, ssX)
            if j == 2:
                @pl.when(k + 1 < NBLK)
                def _():
                    idx_start(k + 1, sN, dN, siN)
            if j < 7:
                s_wait(rY, dC, j, ssY)
                g_start(sC, j + 1, rY, sgY)
            else:
                s_wait(rY, dC, j, ssY)
                @pl.when(k + 1 < NBLK)
                def _():
                    idx_wait(sN, dN, siN)
                    g_start(sN, 0, rY, sgY)

    idx_start(0, sA, dA, siA)
    pltpu.async_copy(rA, agg_sh.at[pI], ssA, add=True)
    pltpu.async_copy(rB, agg_sh.at[pI], ssB, add=True)
    s_wait(rA, dA, 0, ssA)
    idx_wait(sA, dA, siA)
    g_start(sA, 0, rA, sgA)

    @pl.loop(0, NBLK, step=2)
    def _(k):
        do_block(k, sA, dA, siA, sB, dB, siB)
        do_block(k + 1, sB, dB, siB, sA, dA, siA)

    s_wait(rB, dB, 0, ssB)

    plsc.subcore_barrier()
    pltpu.sync_copy(agg_sh.at[pl.ds(t * RPT, RPT)],
                    out_hbm.at[c, pl.ds(t * RPT, RPT)])


_mp_kernel = pl.kernel(_mp_body, **_MP_KW)


def _bn(x, g, b):
    m = jnp.mean(x, axis=0, keepdims=True)
    v = jnp.mean((x - m) ** 2, axis=0, keepdims=True)
    return g * (x - m) / jnp.sqrt(v + 1e-5) + b


def _norm_scale(deg):
    return lax.rsqrt(jnp.maximum(deg, 1.0))


def _tc1_body(deg2_ref, nf_ref, we_ref, be_ref, wc1_ref, x_out, hs_out):
    ns = _norm_scale(deg2_ref[0][:N])
    x = jnp.dot(nf_ref[...], we_ref[...], preferred_element_type=_f32)
    x = x + be_ref[...]
    x_out[...] = x
    hs_out[:N] = jnp.dot(x, wc1_ref[...], preferred_element_type=_f32) * ns
    hs_out[N:] = jnp.zeros((NP - N, D), _f32)


_tc1 = pl.pallas_call(
    _tc1_body,
    out_shape=[
        jax.ShapeDtypeStruct((N, D), _f32),
        jax.ShapeDtypeStruct((NP, D), _f32),
    ],
)


def _tc2_body(agg_ref, deg2_ref, bc1_ref, g1a, be1a, g1b, be1b, x_ref,
              wc2_ref, h_out, hs_out):
    ns = _norm_scale(deg2_ref[0][:N])
    nd = _norm_scale(deg2_ref[1][:N])
    aggf = agg_ref[0][:N] + agg_ref[1][:N]
    h1 = aggf * nd + bc1_ref[...]
    t = jax.nn.relu(_bn(h1, g1a[...], be1a[...]))
    t = jax.nn.relu(_bn(t, g1b[...], be1b[...]))
    h = x_ref[...] + t
    h_out[...] = h
    hs_out[:N] = jnp.dot(h, wc2_ref[...], preferred_element_type=_f32) * ns
    hs_out[N:] = jnp.zeros((NP - N, D), _f32)


_tc2 = pl.pallas_call(
    _tc2_body,
    out_shape=[
        jax.ShapeDtypeStruct((N, D), _f32),
        jax.ShapeDtypeStruct((NP, D), _f32),
    ],
)


def _tc3_body(agg_ref, deg2_ref, bc2, g2a, be2a, g2b, be2b, xr_ref,
              wout, bout, wci, bci, w3a, b3a, g3a, be3a,
              w3b, b3b, g3b, be3b, wcls, bcls, h3d_out, label_out):
    nd = _norm_scale(deg2_ref[1][:N])
    aggf = agg_ref[0][:N] + agg_ref[1][:N]
    h2 = aggf * nd + bc2[...]
    t = jax.nn.relu(_bn(h2, g2a[...], be2a[...]))
    t = jax.nn.relu(_bn(t, g2b[...], be2b[...]))
    h = xr_ref[...] + t
    h3d = jnp.dot(h, wout[...], preferred_element_type=_f32) + bout[...]
    h3d_out[...] = h3d
    pooled = jnp.mean(h3d, axis=0, keepdims=True)
    cc = jnp.dot(pooled, wci[...], preferred_element_type=_f32) + bci[...]
    z = jnp.dot(cc, w3a[...], preferred_element_type=_f32) + b3a[...]
    t2 = jax.nn.relu(_bn(z, g3a[...], be3a[...]))
    z2 = jnp.dot(t2, w3b[...], preferred_element_type=_f32) + b3b[...]
    t3 = jax.nn.relu(_bn(z2, g3b[...], be3b[...]))
    label_out[...] = jnp.dot(t3, wcls[...], preferred_element_type=_f32) + bcls[...]


_tc3 = pl.pallas_call(
    _tc3_body,
    out_shape=[
        jax.ShapeDtypeStruct((N, 3), _f32),
        jax.ShapeDtypeStruct((1, 60), _f32),
    ],
)


def kernel(node_features, edge_index, W_emb, b_emb, W_c1, b_c1, g_1a, be_1a,
           g_1b, be_1b, W_c2, b_c2, g_2a, be_2a, g_2b, be_2b, W_out, b_out,
           W_ci, b_ci, W_3a, b_3a, g_3a, be_3a, W_3b, b_3b, g_3b, be_3b,
           W_cls, b_cls):
    r = lambda v: v.reshape(1, -1)
    z128 = jnp.zeros((RPT, D), _f32)

    pad = N + (jnp.arange(EP - E, dtype=jnp.int32) % (NP - N))
    ei2p = jnp.concatenate([
        edge_index[0], pad, edge_index[1], pad]).reshape(2 * SROWSP, IW)
    o128 = jnp.ones((IW, D), _f32)
    deg2 = _deg_kernel(ei2p, z128, o128).reshape(NCORE, NP, 1)
    x, hs1 = _tc1(deg2, node_features, W_emb, r(b_emb), W_c1)
    agg1 = _mp_kernel(hs1, ei2p, z128)
    h, hs2 = _tc2(agg1, deg2, r(b_c1), r(g_1a), r(be_1a), r(g_1b), r(be_1b),
                  x, W_c2)
    agg2 = _mp_kernel(hs2, ei2p, z128)
    h3d, label = _tc3(agg2, deg2, r(b_c2), r(g_2a), r(be_2a), r(g_2b),
                      r(be_2b), h, W_out, r(b_out), W_ci, r(b_ci),
                      W_3a, r(b_3a), r(g_3a), r(be_3a),
                      W_3b, r(b_3b), r(g_3b), r(be_3b), W_cls, r(b_cls))
    return (h3d, label)


# speedup vs baseline: 1.0068x; 1.0068x over previous
import dataclasses
import functools

import jax
import jax.numpy as jnp
from jax import lax
from jax.experimental import pallas as pl
from jax.experimental.pallas import tpu as pltpu
from jax.experimental.pallas import tpu_sc as plsc

N = 10000
E = 320000
D = 128
H = 64
NCORE = 2
NSUB = 16
IW = 128
SROWS = E // IW
DROWS = SROWS
EP = 327680
SROWSP = EP // IW
MROWSP = SROWSP // NCORE
TROWS = MROWSP // NSUB
NBLK = TROWS // 8
NP = 10240
RPT = NP // NSUB

_mesh = plsc.VectorSubcoreMesh(core_axis_name="c", subcore_axis_name="s")
_f32 = jnp.float32


DTROWS = SROWSP // NSUB
HR = NP // IW
RED = HR // 8

_deg_cp = pltpu.CompilerParams()
if "needs_layout_passes" in pltpu.CompilerParams.__dataclass_fields__:
    _deg_cp = dataclasses.replace(_deg_cp, needs_layout_passes=False)

_DEG_KW = dict(
    mesh=_mesh,
    compiler_params=_deg_cp,
    out_type=jax.ShapeDtypeStruct((NCORE, HR, IW), _f32),
    scratch_types=[
        pltpu.VMEM((DTROWS, IW), jnp.int32),
        pltpu.VMEM((HR, IW), _f32),
        pltpu.VMEM((8, IW), _f32),
        pltpu.VMEM((8, IW), _f32),
        pltpu.VMEM((16,), _f32),
        pltpu.VMEM_SHARED((NSUB * HR, IW), _f32),
    ],
)


def _deg_body(ei2_hbm, z128_hbm, o128_hbm, out_hbm,
              idx_v, hist_v, acc_v, tmp_v, one_v, hist_sh):
    c = lax.axis_index("c")
    t = lax.axis_index("s")

    pltpu.sync_copy(z128_hbm.at[pl.ds(0, HR)], hist_v)
    pltpu.sync_copy(ei2_hbm.at[pl.ds(c * SROWSP + t * DTROWS, DTROWS)], idx_v)
    pltpu.sync_copy(o128_hbm.at[0, pl.ds(0, 16)], one_v)
    ones16 = one_v[...]

    @pl.loop(0, DTROWS)
    def _(j):
        for l in range(IW // 16):
            idx16 = idx_v[j, pl.ds(l * 16, 16)]
            ri = lax.shift_right_logical(idx16, 7)
            ci = lax.bitwise_and(idx16, 127)
            plsc.addupdate_scatter(hist_v, [ri, ci], ones16)

    pltpu.sync_copy(hist_v, hist_sh.at[pl.ds(t * HR, HR)])
    plsc.subcore_barrier()

    @pl.when(t < RED)
    def _():
        pltpu.sync_copy(hist_sh.at[pl.ds(t * 8, 8)], acc_v)

        @pl.loop(1, NSUB)
        def _(h):
            pltpu.sync_copy(hist_sh.at[pl.ds(h * HR + t * 8, 8)], tmp_v)
            for i in range(8):
                for l in range(IW // 16):
                    sl = (i, pl.ds(l * 16, 16))
                    acc_v[sl] = acc_v[sl] + tmp_v[sl]

        pltpu.sync_copy(acc_v, out_hbm.at[c, pl.ds(t * 8, 8)])


_deg_kernel = pl.kernel(_deg_body, **_DEG_KW)


_MP_KW = dict(
    mesh=_mesh,
    out_type=jax.ShapeDtypeStruct((NCORE, NP, D), _f32),
    scratch_types=[
        pltpu.VMEM((8, IW), jnp.int32),
        pltpu.VMEM((8, IW), jnp.int32),
        pltpu.VMEM((8, IW), jnp.int32),
        pltpu.VMEM((8, IW), jnp.int32),
        pltpu.VMEM((IW, D), _f32),
        pltpu.VMEM((IW, D), _f32),
        pltpu.VMEM_SHARED((NP, D), _f32),
        pltpu.SemaphoreType.DMA,
        pltpu.SemaphoreType.DMA,
        pltpu.SemaphoreType.DMA,
        pltpu.SemaphoreType.DMA,
    ],
)


def _mp_body(tab_hbm, ei2_hbm, z128_hbm, out_hbm,
             sA, dA, sB, dB, rA, rB, agg_sh, siA, siB, sgA, sgB):
    c = lax.axis_index("c")
    t = lax.axis_index("s")

    pltpu.sync_copy(z128_hbm, agg_sh.at[pl.ds(t * RPT, RPT)])
    plsc.subcore_barrier()

    base = c * MROWSP + t * TROWS

    def idx_start(k, s_ref, d_ref, sem):
        r0 = base + 8 * k
        pltpu.async_copy(ei2_hbm.at[pl.ds(r0, 8)], s_ref, sem)
        pltpu.async_copy(ei2_hbm.at[pl.ds(SROWSP + r0, 8)], d_ref, sem)

    def idx_wait(s_ref, d_ref, sem):
        pltpu.make_async_copy(ei2_hbm.at[pl.ds(0, 8)], s_ref, sem).wait()
        pltpu.make_async_copy(ei2_hbm.at[pl.ds(0, 8)], d_ref, sem).wait()

    def g_start(s_ref, j, r_ref, sem):
        pltpu.async_copy(tab_hbm.at[s_ref.at[j]], r_ref, sem)

    def g_wait(s_ref, j, r_ref, sem):
        pltpu.make_async_copy(tab_hbm.at[s_ref.at[j]], r_ref, sem).wait()

    def do_block(k, sC, dC, siC, sN, dN, siN):
        @pl.when(k + 1 < NBLK)
        def _():
            idx_start(k + 1, sN, dN, siN)

        for j in range(8):
            if j % 2 == 0:
                rX, sgX, rY, sgY = rA, sgA, rB, sgB
            else:
                rX, sgX, rY, sgY = rB, sgB, rA, sgA
            g_wait(sC, j, rX, sgX)
            if j < 7:
                g_start(sC, j + 1, rY, sgY)
            else:
                @pl.when(k + 1 < NBLK)
                def _():
                    idx_wait(sN, dN, siN)
                    g_start(sN, 0, rY, sgY)
            pltpu.sync_copy(rX, agg_sh.at[dC.at[j]], add=True)

    idx_start(0, sA, dA, siA)
    idx_wait(sA, dA, siA)
    g_start(sA, 0, rA, sgA)

    @pl.loop(0, NBLK, step=2)
    def _(k):
        do_block(k, sA, dA, siA, sB, dB, siB)
        do_block(k + 1, sB, dB, siB, sA, dA, siA)

    plsc.subcore_barrier()
    pltpu.sync_copy(agg_sh.at[pl.ds(t * RPT, RPT)],
                    out_hbm.at[c, pl.ds(t * RPT, RPT)])


_mp_kernel = pl.kernel(_mp_body, **_MP_KW)


def _bn(x, g, b):
    m = jnp.mean(x, axis=0, keepdims=True)
    v = jnp.mean((x - m) ** 2, axis=0, keepdims=True)
    return g * (x - m) / jnp.sqrt(v + 1e-5) + b


def _norm_scale(deg):
    return lax.rsqrt(jnp.maximum(deg, 1.0))


def _tc1_body(deg2_ref, nf_ref, we_ref, be_ref, wc1_ref, x_out, hs_out):
    ns = _norm_scale(deg2_ref[0][:N])
    x = jnp.dot(nf_ref[...], we_ref[...], preferred_element_type=_f32)
    x = x + be_ref[...]
    x_out[...] = x
    hs_out[:N] = jnp.dot(x, wc1_ref[...], preferred_element_type=_f32) * ns
    hs_out[N:] = jnp.zeros((NP - N, D), _f32)


_tc1 = pl.pallas_call(
    _tc1_body,
    out_shape=[
        jax.ShapeDtypeStruct((N, D), _f32),
        jax.ShapeDtypeStruct((NP, D), _f32),
    ],
)


def _tc2_body(agg_ref, deg2_ref, bc1_ref, g1a, be1a, g1b, be1b, x_ref,
              wc2_ref, h_out, hs_out):
    ns = _norm_scale(deg2_ref[0][:N])
    nd = _norm_scale(deg2_ref[1][:N])
    aggf = agg_ref[0][:N] + agg_ref[1][:N]
    h1 = aggf * nd + bc1_ref[...]
    t = jax.nn.relu(_bn(h1, g1a[...], be1a[...]))
    t = jax.nn.relu(_bn(t, g1b[...], be1b[...]))
    h = x_ref[...] + t
    h_out[...] = h
    hs_out[:N] = jnp.dot(h, wc2_ref[...], preferred_element_type=_f32) * ns
    hs_out[N:] = jnp.zeros((NP - N, D), _f32)


_tc2 = pl.pallas_call(
    _tc2_body,
    out_shape=[
        jax.ShapeDtypeStruct((N, D), _f32),
        jax.ShapeDtypeStruct((NP, D), _f32),
    ],
)


def _tc3_body(agg_ref, deg2_ref, bc2, g2a, be2a, g2b, be2b, xr_ref,
              wout, bout, wci, bci, w3a, b3a, g3a, be3a,
              w3b, b3b, g3b, be3b, wcls, bcls, h3d_out, label_out):
    nd = _norm_scale(deg2_ref[1][:N])
    aggf = agg_ref[0][:N] + agg_ref[1][:N]
    h2 = aggf * nd + bc2[...]
    t = jax.nn.relu(_bn(h2, g2a[...], be2a[...]))
    t = jax.nn.relu(_bn(t, g2b[...], be2b[...]))
    h = xr_ref[...] + t
    h3d = jnp.dot(h, wout[...], preferred_element_type=_f32) + bout[...]
    h3d_out[...] = h3d
    pooled = jnp.mean(h3d, axis=0, keepdims=True)
    cc = jnp.dot(pooled, wci[...], preferred_element_type=_f32) + bci[...]
    z = jnp.dot(cc, w3a[...], preferred_element_type=_f32) + b3a[...]
    t2 = jax.nn.relu(_bn(z, g3a[...], be3a[...]))
    z2 = jnp.dot(t2, w3b[...], preferred_element_type=_f32) + b3b[...]
    t3 = jax.nn.relu(_bn(z2, g3b[...], be3b[...]))
    label_out[...] = jnp.dot(t3, wcls[...], preferred_element_type=_f32) + bcls[...]


_tc3 = pl.pallas_call(
    _tc3_body,
    out_shape=[
        jax.ShapeDtypeStruct((N, 3), _f32),
        jax.ShapeDtypeStruct((1, 60), _f32),
    ],
)


def kernel(node_features, edge_index, W_emb, b_emb, W_c1, b_c1, g_1a, be_1a,
           g_1b, be_1b, W_c2, b_c2, g_2a, be_2a, g_2b, be_2b, W_out, b_out,
           W_ci, b_ci, W_3a, b_3a, g_3a, be_3a, W_3b, b_3b, g_3b, be_3b,
           W_cls, b_cls):
    r = lambda v: v.reshape(1, -1)
    z128 = jnp.zeros((RPT, D), _f32)

    pad = N + (jnp.arange(EP - E, dtype=jnp.int32) % (NP - N))
    ei2p = jnp.concatenate([
        edge_index[0], pad, edge_index[1], pad]).reshape(2 * SROWSP, IW)
    o128 = jnp.ones((IW, D), _f32)
    deg2 = _deg_kernel(ei2p, z128, o128).reshape(NCORE, NP, 1)
    x, hs1 = _tc1(deg2, node_features, W_emb, r(b_emb), W_c1)
    agg1 = _mp_kernel(hs1, ei2p, z128)
    h, hs2 = _tc2(agg1, deg2, r(b_c1), r(g_1a), r(be_1a), r(g_1b), r(be_1b),
                  x, W_c2)
    agg2 = _mp_kernel(hs2, ei2p, z128)
    h3d, label = _tc3(agg2, deg2, r(b_c2), r(g_2a), r(be_2a), r(g_2b),
                      r(be_2b), h, W_out, r(b_out), W_ci, r(b_ci),
                      W_3a, r(b_3a), r(g_3a), r(be_3a),
                      W_3b, r(b_3b), r(g_3b), r(be_3b), W_cls, r(b_cls))
    return (h3d, label)
